# baseline (speedup 1.0000x reference)
import functools

import jax
import jax.numpy as jnp
from jax import lax
from jax.experimental import pallas as pl
from jax.experimental.pallas import tpu as pltpu
from jax.experimental.pallas import tpu_sc as plsc

NUM_CORES = 2
NUM_SUBCORES = 16
NW = NUM_CORES * NUM_SUBCORES

B, L, C, D = 1024, 50, 26, 64
N = B * L
PER_W = N // NW
P = 80
NBLK = PER_W // P
LG = D // 16


NBUF = 4


def _body(table, idx, out, idx_raw, idx_v, acc, *sems):
    sem_g = sems[0:NBUF]
    sem_o = sems[NBUF : 2 * NBUF]

    cid = lax.axis_index("c")
    sid = lax.axis_index("s")
    wid = sid * NUM_CORES + cid
    obase = wid * PER_W

    pltpu.sync_copy(idx.at[pl.ds(wid * PER_W * C, PER_W * C)], idx_raw)
    lane26 = lax.iota(jnp.int32, 16) * C

    def tr_body(g, carry):
        base = g * (16 * C)
        for j in range(C):
            vec = plsc.load_gather(idx_raw, [base + j + lane26])
            idx_v[j, pl.ds(g * 16, 16)] = vec
        return carry

    lax.fori_loop(0, PER_W // 16, tr_body, 0)

    zero = jnp.zeros((16,), jnp.float32)

    def zero_buf(bb):
        for r in range(P):
            for dg in range(LG):
                acc[bb, r, pl.ds(dg * 16, 16)] = zero

    def fire_block(f, bb):
        col0 = f * P
        for j in range(C):
            pltpu.async_copy(
                table.at[idx_v.at[j, pl.ds(col0, P)]],
                acc.at[bb],
                sem_g[bb],
                add=True,
            )

    def drain_block(f, bb):
        col0 = f * P
        for j in range(C):
            pltpu.make_async_copy(
                table.at[idx_v.at[j, pl.ds(col0, P)]],
                acc.at[bb],
                sem_g[bb],
            ).wait()

    def fire_out(f, bb):
        pltpu.async_copy(
            acc.at[bb], out.at[pl.ds(obase + f * P, P)], sem_o[bb]
        )

    def wait_out(f, bb):
        pltpu.make_async_copy(
            acc.at[bb], out.at[pl.ds(obase + f * P, P)], sem_o[bb]
        ).wait()

    for g in range(2):
        zero_buf(g)
        fire_block(g, g)

    def body(i, carry):
        for bb in range(NBUF):
            f = NBUF * i + bb - 2
            fg = f + 2
            b_drain = (bb - 2) % NBUF

            @pl.when(jnp.logical_and(f >= 0, f < NBLK))
            def _():
                drain_block(f, b_drain)
                fire_out(f, b_drain)

            @pl.when(jnp.logical_and(fg >= 2, fg < NBLK))
            def _():
                prev = fg - NBUF

                @pl.when(prev >= 0)
                def _():
                    wait_out(prev, bb)

                zero_buf(bb)
                fire_block(fg, bb)

        return carry

    lax.fori_loop(0, NBLK // NBUF + 1, body, 0)
    for f in range(NBLK - NBUF, NBLK):
        wait_out(f, f % NBUF)


_embed_sum = functools.partial(
    pl.kernel,
    mesh=plsc.VectorSubcoreMesh(
        core_axis_name="c", subcore_axis_name="s",
        num_cores=NUM_CORES, num_subcores=NUM_SUBCORES,
    ),
    out_type=jax.ShapeDtypeStruct((N, D), jnp.float32),
    scratch_types=[
        pltpu.VMEM((PER_W * C,), jnp.int32),
        pltpu.VMEM((C, PER_W), jnp.int32),
        pltpu.VMEM((NBUF, P, D), jnp.float32),
    ]
    + [pltpu.SemaphoreType.DMA] * (2 * NBUF),
    compiler_params=pltpu.CompilerParams(
        use_tc_tiling_on_sc=False, needs_layout_passes=False
    ),
)(_body)


@jax.jit
def kernel(node_idx, clause_enc):
    idx1d = node_idx.astype(jnp.int32).reshape(-1)
    out = _embed_sum(clause_enc, idx1d)
    return out.reshape(B, L, D)


# speedup vs baseline: 1.1080x; 1.1080x over previous
import functools

import jax
import jax.numpy as jnp
from jax import lax
from jax.experimental import pallas as pl
from jax.experimental.pallas import tpu as pltpu
from jax.experimental.pallas import tpu_sc as plsc

NUM_CORES = 2
NUM_SUBCORES = 16
NW = NUM_CORES * NUM_SUBCORES

B, L, C, D = 1024, 50, 26, 64
N = B * L
PER_W = N // NW
P = 80
NBLK = PER_W // P
LG = D // 16


NBUF = 2


def _body(table, idx, out, idx_v, acc, *sems):
    sem_g = sems[0:NBUF]
    sem_o = sems[NBUF : 2 * NBUF]

    cid = lax.axis_index("c")
    sid = lax.axis_index("s")
    wid = sid * NUM_CORES + cid
    obase = wid * PER_W

    pltpu.sync_copy(idx.at[:, pl.ds(wid * PER_W, PER_W)], idx_v)

    zero = jnp.zeros((16,), jnp.float32)

    def zero_buf(bb):
        for r in range(P):
            for dg in range(LG):
                acc[bb, r, pl.ds(dg * 16, 16)] = zero

    def fire_block(f, bb):
        col0 = f * P
        for j in range(C):
            pltpu.async_copy(
                table.at[idx_v.at[j, pl.ds(col0, P)]],
                acc.at[bb],
                sem_g[bb],
                add=True,
            )

    def drain_block(f, bb):
        col0 = f * P
        for j in range(C):
            pltpu.make_async_copy(
                table.at[idx_v.at[j, pl.ds(col0, P)]],
                acc.at[bb],
                sem_g[bb],
            ).wait()

    def fire_out(f, bb):
        pltpu.async_copy(
            acc.at[bb], out.at[pl.ds(obase + f * P, P)], sem_o[bb]
        )

    def wait_out(f, bb):
        pltpu.make_async_copy(
            acc.at[bb], out.at[pl.ds(obase + f * P, P)], sem_o[bb]
        ).wait()

    for g in range(2):
        zero_buf(g)
        fire_block(g, g)

    def body(i, carry):
        for bb in range(NBUF):
            f = NBUF * i + bb - 2
            fg = f + 2
            b_drain = (bb - 2) % NBUF

            @pl.when(jnp.logical_and(f >= 0, f < NBLK))
            def _():
                drain_block(f, b_drain)
                fire_out(f, b_drain)

            @pl.when(jnp.logical_and(fg >= 2, fg < NBLK))
            def _():
                prev = fg - NBUF

                @pl.when(prev >= 0)
                def _():
                    wait_out(prev, bb)

                zero_buf(bb)
                fire_block(fg, bb)

        return carry

    lax.fori_loop(0, NBLK // NBUF + 1, body, 0)
    for f in range(NBLK - NBUF, NBLK):
        wait_out(f, f % NBUF)


_embed_sum = functools.partial(
    pl.kernel,
    mesh=plsc.VectorSubcoreMesh(
        core_axis_name="c", subcore_axis_name="s",
        num_cores=NUM_CORES, num_subcores=NUM_SUBCORES,
    ),
    out_type=jax.ShapeDtypeStruct((N, D), jnp.float32),
    scratch_types=[
        pltpu.VMEM((C, PER_W), jnp.int32),
        pltpu.VMEM((NBUF, P, D), jnp.float32),
    ]
    + [pltpu.SemaphoreType.DMA] * (2 * NBUF),
    compiler_params=pltpu.CompilerParams(use_tc_tiling_on_sc=False),
)(_body)


@jax.jit
def kernel(node_idx, clause_enc):
    idx_t = node_idx.astype(jnp.int32).reshape(N, C).T
    out = _embed_sum(clause_enc, idx_t)
    return out.reshape(B, L, D)
